# baseline (speedup 1.0000x reference)
import functools

import jax
import jax.numpy as jnp
from jax import lax
from jax.experimental import pallas as pl
from jax.experimental.pallas import tpu as pltpu
from jax.experimental.pallas import tpu_sc as plsc

_NC = 2
_NS = 16
_NW = _NC * _NS
_LANES = 16


def _sc_gather_mul(F0, F1, idx0, idx1):
    B = idx0.shape[0]
    K = F0.shape[1]
    b_per_w = B // _NW
    mesh = plsc.VectorSubcoreMesh(core_axis_name="c", subcore_axis_name="s")

    @functools.partial(
        pl.kernel,
        mesh=mesh,
        out_type=jax.ShapeDtypeStruct((B, K), jnp.float32),
        scratch_types=[
            pltpu.VMEM((b_per_w,), jnp.int32),
            pltpu.VMEM((b_per_w,), jnp.int32),
            pltpu.VMEM((b_per_w, K), jnp.float32),
            pltpu.VMEM((b_per_w, K), jnp.float32),
            pltpu.SemaphoreType.DMA,
            pltpu.SemaphoreType.DMA,
        ],
    )
    def sc_kernel(idx0_hbm, idx1_hbm, f0_hbm, f1_hbm, out_hbm,
                  i0_v, i1_v, r0_v, r1_v, sem0, sem1):
        wid = lax.axis_index("s") * _NC + lax.axis_index("c")
        base = wid * b_per_w
        pltpu.sync_copy(idx0_hbm.at[pl.ds(base, b_per_w)], i0_v)
        pltpu.sync_copy(idx1_hbm.at[pl.ds(base, b_per_w)], i1_v)
        c0 = pltpu.async_copy(f0_hbm.at[i0_v], r0_v, sem0)
        c1 = pltpu.async_copy(f1_hbm.at[i1_v], r1_v, sem1)
        c0.wait()
        c1.wait()

        def row_body(r, carry):
            for j in range(K // _LANES):
                sl = pl.ds(j * _LANES, _LANES)
                r0_v[r, sl] = r0_v[r, sl] * r1_v[r, sl]
            return carry

        lax.fori_loop(0, b_per_w, row_body, 0, unroll=2)
        pltpu.sync_copy(r0_v, out_hbm.at[pl.ds(base, b_per_w)])

    return sc_kernel(idx0, idx1, F0, F1)


def _tc_matmul_chunk(prod_c, F2, acc, chunk, n_chunks):
    Bc, K = prod_c.shape
    N = F2.shape[0]
    B = Bc * n_chunks
    BLK = 512
    nblk = Bc // BLK

    def mm_body(p_ref, f2_ref, *rest):
        o_ref = rest[-1]
        o_ref[...] = lax.dot_general(
            p_ref[...], f2_ref[...],
            (((1,), (1,)), ((), ())),
            preferred_element_type=jnp.float32,
        )

    in_specs = [
        pl.BlockSpec((BLK, K), lambda i: (i, 0)),
        pl.BlockSpec((N, K), lambda i: (0, 0)),
    ]
    operands = [prod_c, F2]
    kwargs = {}
    if acc is not None:
        in_specs.append(pl.BlockSpec(memory_space=pl.ANY))
        operands.append(acc)
        kwargs["input_output_aliases"] = {2: 0}

    off = chunk * nblk
    return pl.pallas_call(
        mm_body,
        grid=(nblk,),
        in_specs=in_specs,
        out_specs=pl.BlockSpec((BLK, N), lambda i: (i + off, 0)),
        out_shape=jax.ShapeDtypeStruct((B, N), jnp.float32),
        **kwargs,
    )(*operands)


_N_CHUNKS = 2


def kernel(indices, F0, F1, F2, log_sigma):
    idx0 = indices[:, 0].astype(jnp.int32)
    idx1 = indices[:, 1].astype(jnp.int32)
    B = idx0.shape[0]
    Bc = B // _N_CHUNKS
    prods = [
        _sc_gather_mul(F0, F1,
                       lax.dynamic_slice_in_dim(idx0, c * Bc, Bc),
                       lax.dynamic_slice_in_dim(idx1, c * Bc, Bc))
        for c in range(_N_CHUNKS)
    ]
    res = None
    for c, prod_c in enumerate(prods):
        res = _tc_matmul_chunk(prod_c, F2, res, c, _N_CHUNKS)
    return (res, log_sigma)


# speedup vs baseline: 1.4514x; 1.4514x over previous
import functools

import jax
import jax.numpy as jnp
from jax import lax
from jax.experimental import pallas as pl
from jax.experimental.pallas import tpu as pltpu
from jax.experimental.pallas import tpu_sc as plsc

_NC = 2
_NS = 16
_NW = _NC * _NS
_LANES = 16


def _sc_gather_mul(F0, F1, idx0, idx1):
    B = idx0.shape[0]
    K = F0.shape[1]
    b_per_w = B // _NW
    mesh = plsc.VectorSubcoreMesh(core_axis_name="c", subcore_axis_name="s")

    @functools.partial(
        pl.kernel,
        mesh=mesh,
        out_type=jax.ShapeDtypeStruct((B, K), jnp.float32),
        scratch_types=[
            pltpu.VMEM((b_per_w,), jnp.int32),
            pltpu.VMEM((b_per_w,), jnp.int32),
            pltpu.VMEM((b_per_w, K), jnp.float32),
            pltpu.VMEM((b_per_w, K), jnp.float32),
            pltpu.SemaphoreType.DMA,
            pltpu.SemaphoreType.DMA,
        ],
    )
    def sc_kernel(idx0_hbm, idx1_hbm, f0_hbm, f1_hbm, out_hbm,
                  i0_v, i1_v, r0_v, r1_v, sem0, sem1):
        wid = lax.axis_index("s") * _NC + lax.axis_index("c")
        base = wid * b_per_w
        pltpu.sync_copy(idx0_hbm.at[pl.ds(base, b_per_w)], i0_v)
        pltpu.sync_copy(idx1_hbm.at[pl.ds(base, b_per_w)], i1_v)
        c0 = pltpu.async_copy(f0_hbm.at[i0_v], r0_v, sem0)
        c1 = pltpu.async_copy(f1_hbm.at[i1_v], r1_v, sem1)
        c0.wait()
        c1.wait()

        def row_body(r, carry):
            for j in range(K // _LANES):
                sl = pl.ds(j * _LANES, _LANES)
                r0_v[r, sl] = r0_v[r, sl] * r1_v[r, sl]
            return carry

        lax.fori_loop(0, b_per_w, row_body, 0, unroll=2)
        pltpu.sync_copy(r0_v, out_hbm.at[pl.ds(base, b_per_w)])

    return sc_kernel(idx0, idx1, F0, F1)


def _tc_matmul_t(prod, F2):
    B, K = prod.shape
    N = F2.shape[0]
    BLK = 512

    def mm_body(f2_ref, p_ref, o_ref):
        o_ref[...] = lax.dot_general(
            f2_ref[...], p_ref[...],
            (((1,), (1,)), ((), ())),
            preferred_element_type=jnp.float32,
        )

    return pl.pallas_call(
        mm_body,
        grid=(B // BLK,),
        in_specs=[
            pl.BlockSpec((N, K), lambda i: (0, 0)),
            pl.BlockSpec((BLK, K), lambda i: (i, 0)),
        ],
        out_specs=pl.BlockSpec((N, BLK), lambda i: (0, i)),
        out_shape=jax.ShapeDtypeStruct((N, B), jnp.float32),
    )(F2, prod)


def kernel(indices, F0, F1, F2, log_sigma):
    idx0 = indices[:, 0].astype(jnp.int32)
    idx1 = indices[:, 1].astype(jnp.int32)
    prod = _sc_gather_mul(F0, F1, idx0, idx1)
    res_t = _tc_matmul_t(prod, F2)
    return (res_t.T, log_sigma)


# speedup vs baseline: 1.4739x; 1.0155x over previous
import functools

import jax
import jax.numpy as jnp
from jax import lax
from jax.experimental import pallas as pl
from jax.experimental.pallas import tpu as pltpu
from jax.experimental.pallas import tpu_sc as plsc

_NC = 2
_NS = 16
_NW = _NC * _NS
_LANES = 16
_NCHUNK = 4


def _sc_gather_mul(F0, F1, idx0, idx1):
    B = idx0.shape[0]
    K = F0.shape[1]
    b_per_w = B // _NW
    rows_c = b_per_w // _NCHUNK
    mesh = plsc.VectorSubcoreMesh(core_axis_name="c", subcore_axis_name="s")

    @functools.partial(
        pl.kernel,
        mesh=mesh,
        out_type=jax.ShapeDtypeStruct((B, K), jnp.float32),
        scratch_types=[
            pltpu.VMEM((b_per_w,), jnp.int32),
            pltpu.VMEM((b_per_w,), jnp.int32),
            pltpu.VMEM((b_per_w, K), jnp.float32),
            pltpu.VMEM((b_per_w, K), jnp.float32),
            pltpu.SemaphoreType.DMA,
            pltpu.SemaphoreType.DMA,
        ]
        + [pltpu.SemaphoreType.DMA] * (2 * _NCHUNK)
        + [pltpu.SemaphoreType.DMA] * _NCHUNK,
    )
    def sc_kernel(idx0_hbm, idx1_hbm, f0_hbm, f1_hbm, out_hbm,
                  i0_v, i1_v, r0_v, r1_v, sem_i0, sem_i1, *sems):
        g_sems = sems[: 2 * _NCHUNK]
        w_sems = sems[2 * _NCHUNK:]
        wid = lax.axis_index("s") * _NC + lax.axis_index("c")
        base = wid * b_per_w
        ci0 = pltpu.async_copy(idx0_hbm.at[pl.ds(base, b_per_w)], i0_v, sem_i0)
        ci1 = pltpu.async_copy(idx1_hbm.at[pl.ds(base, b_per_w)], i1_v, sem_i1)
        ci0.wait()
        ci1.wait()

        def issue_gathers(c):
            lo = c * rows_c
            g0 = pltpu.async_copy(f0_hbm.at[i0_v.at[pl.ds(lo, rows_c)]],
                                  r0_v.at[pl.ds(lo, rows_c)], g_sems[2 * c])
            g1 = pltpu.async_copy(f1_hbm.at[i1_v.at[pl.ds(lo, rows_c)]],
                                  r1_v.at[pl.ds(lo, rows_c)], g_sems[2 * c + 1])
            return g0, g1

        pending = issue_gathers(0)
        writes = []
        for c in range(_NCHUNK):
            nxt = issue_gathers(c + 1) if c + 1 < _NCHUNK else None
            pending[0].wait()
            pending[1].wait()
            pending = nxt
            lo = c * rows_c

            def row_body(r, carry):
                for j in range(K // _LANES):
                    sl = pl.ds(j * _LANES, _LANES)
                    r0_v[r, sl] = r0_v[r, sl] * r1_v[r, sl]
                return carry

            lax.fori_loop(lo, lo + rows_c, row_body, 0, unroll=2)
            writes.append(pltpu.async_copy(
                r0_v.at[pl.ds(lo, rows_c)],
                out_hbm.at[pl.ds(base + lo, rows_c)], w_sems[c]))
        for w in writes:
            w.wait()

    return sc_kernel(idx0, idx1, F0, F1)


def _tc_matmul_t(prod, F2):
    B, K = prod.shape
    N = F2.shape[0]
    BLK = 512

    def mm_body(f2_ref, p_ref, o_ref):
        o_ref[...] = lax.dot_general(
            f2_ref[...], p_ref[...],
            (((1,), (1,)), ((), ())),
            preferred_element_type=jnp.float32,
        )

    return pl.pallas_call(
        mm_body,
        grid=(B // BLK,),
        in_specs=[
            pl.BlockSpec((N, K), lambda i: (0, 0)),
            pl.BlockSpec((BLK, K), lambda i: (i, 0)),
        ],
        out_specs=pl.BlockSpec((N, BLK), lambda i: (0, i)),
        out_shape=jax.ShapeDtypeStruct((N, B), jnp.float32),
    )(F2, prod)


def kernel(indices, F0, F1, F2, log_sigma):
    idx0 = indices[:, 0].astype(jnp.int32)
    idx1 = indices[:, 1].astype(jnp.int32)
    prod = _sc_gather_mul(F0, F1, idx0, idx1)
    res_t = _tc_matmul_t(prod, F2)
    return (res_t.T, log_sigma)


# speedup vs baseline: 1.4767x; 1.0019x over previous
import functools

import jax
import jax.numpy as jnp
from jax import lax
from jax.experimental import pallas as pl
from jax.experimental.pallas import tpu as pltpu
from jax.experimental.pallas import tpu_sc as plsc

_NC = 2
_NS = 16
_NW = _NC * _NS
_LANES = 16
_NCHUNK = 8


def _sc_gather_mul(F0, F1, idx0, idx1):
    B = idx0.shape[0]
    K = F0.shape[1]
    b_per_w = B // _NW
    rows_c = b_per_w // _NCHUNK
    mesh = plsc.VectorSubcoreMesh(core_axis_name="c", subcore_axis_name="s")

    @functools.partial(
        pl.kernel,
        mesh=mesh,
        out_type=jax.ShapeDtypeStruct((B, K), jnp.float32),
        scratch_types=[
            pltpu.VMEM((b_per_w,), jnp.int32),
            pltpu.VMEM((b_per_w,), jnp.int32),
            pltpu.VMEM((b_per_w, K), jnp.float32),
            pltpu.VMEM((b_per_w, K), jnp.float32),
            pltpu.SemaphoreType.DMA,
            pltpu.SemaphoreType.DMA,
        ]
        + [pltpu.SemaphoreType.DMA] * (2 * _NCHUNK)
        + [pltpu.SemaphoreType.DMA] * _NCHUNK,
    )
    def sc_kernel(idx0_hbm, idx1_hbm, f0_hbm, f1_hbm, out_hbm,
                  i0_v, i1_v, r0_v, r1_v, sem_i0, sem_i1, *sems):
        g_sems = sems[: 2 * _NCHUNK]
        w_sems = sems[2 * _NCHUNK:]
        wid = lax.axis_index("s") * _NC + lax.axis_index("c")
        base = wid * b_per_w
        ci0 = pltpu.async_copy(idx0_hbm.at[pl.ds(base, b_per_w)], i0_v, sem_i0)
        ci1 = pltpu.async_copy(idx1_hbm.at[pl.ds(base, b_per_w)], i1_v, sem_i1)
        ci0.wait()
        ci1.wait()

        def issue_gathers(c):
            lo = c * rows_c
            g0 = pltpu.async_copy(f0_hbm.at[i0_v.at[pl.ds(lo, rows_c)]],
                                  r0_v.at[pl.ds(lo, rows_c)], g_sems[2 * c])
            g1 = pltpu.async_copy(f1_hbm.at[i1_v.at[pl.ds(lo, rows_c)]],
                                  r1_v.at[pl.ds(lo, rows_c)], g_sems[2 * c + 1])
            return g0, g1

        pending = issue_gathers(0)
        writes = []
        for c in range(_NCHUNK):
            nxt = issue_gathers(c + 1) if c + 1 < _NCHUNK else None
            pending[0].wait()
            pending[1].wait()
            pending = nxt
            lo = c * rows_c

            def row_body(r, carry):
                for j in range(K // _LANES):
                    sl = pl.ds(j * _LANES, _LANES)
                    r0_v[r, sl] = r0_v[r, sl] * r1_v[r, sl]
                return carry

            lax.fori_loop(lo, lo + rows_c, row_body, 0, unroll=2)
            writes.append(pltpu.async_copy(
                r0_v.at[pl.ds(lo, rows_c)],
                out_hbm.at[pl.ds(base + lo, rows_c)], w_sems[c]))
        for w in writes:
            w.wait()

    return sc_kernel(idx0, idx1, F0, F1)


def _tc_matmul_t(prod, F2):
    B, K = prod.shape
    N = F2.shape[0]
    BLK = 1024

    def mm_body(f2_ref, p_ref, o_ref):
        o_ref[...] = lax.dot_general(
            f2_ref[...], p_ref[...],
            (((1,), (1,)), ((), ())),
            preferred_element_type=jnp.float32,
        )

    return pl.pallas_call(
        mm_body,
        grid=(B // BLK,),
        in_specs=[
            pl.BlockSpec((N, K), lambda i: (0, 0)),
            pl.BlockSpec((BLK, K), lambda i: (i, 0)),
        ],
        out_specs=pl.BlockSpec((N, BLK), lambda i: (0, i)),
        out_shape=jax.ShapeDtypeStruct((N, B), jnp.float32),
    )(F2, prod)


def kernel(indices, F0, F1, F2, log_sigma):
    idx0 = indices[:, 0].astype(jnp.int32)
    idx1 = indices[:, 1].astype(jnp.int32)
    prod = _sc_gather_mul(F0, F1, idx0, idx1)
    res_t = _tc_matmul_t(prod, F2)
    return (res_t.T, log_sigma)


# speedup vs baseline: 1.5065x; 1.0202x over previous
import functools

import jax
import jax.numpy as jnp
from jax import lax
from jax.experimental import pallas as pl
from jax.experimental.pallas import tpu as pltpu
from jax.experimental.pallas import tpu_sc as plsc

_NC = 2
_NS = 16
_NW = _NC * _NS
_LANES = 16
_NCHUNK = 8


def _vreg_gather(v, c):
    return lax.gather(
        v, c[:, None],
        dimension_numbers=lax.GatherDimensionNumbers(
            offset_dims=(), collapsed_slice_dims=(0,), start_index_map=(0,)),
        slice_sizes=(1,),
        mode=lax.GatherScatterMode.PROMISE_IN_BOUNDS)


def _sc_gather_mul(F0, F1, idx_flat):
    B = idx_flat.shape[0] // 2
    K = F0.shape[1]
    b_per_w = B // _NW
    rows_c = b_per_w // _NCHUNK
    assert rows_c == _LANES
    mesh = plsc.VectorSubcoreMesh(core_axis_name="c", subcore_axis_name="s")

    @functools.partial(
        pl.kernel,
        mesh=mesh,
        out_type=jax.ShapeDtypeStruct((B, K), jnp.float32),
        scratch_types=[
            pltpu.VMEM((2 * b_per_w,), jnp.int32),
            pltpu.VMEM((b_per_w,), jnp.int32),
            pltpu.VMEM((b_per_w,), jnp.int32),
            pltpu.VMEM((b_per_w, K), jnp.float32),
            pltpu.VMEM((b_per_w, K), jnp.float32),
            pltpu.SemaphoreType.DMA,
        ]
        + [pltpu.SemaphoreType.DMA] * (2 * _NCHUNK)
        + [pltpu.SemaphoreType.DMA] * _NCHUNK,
    )
    def sc_kernel(idx_hbm, f0_hbm, f1_hbm, out_hbm,
                  ia_v, i0_v, i1_v, r0_v, r1_v, sem_i, *sems):
        g_sems = sems[: 2 * _NCHUNK]
        w_sems = sems[2 * _NCHUNK:]
        wid = lax.axis_index("s") * _NC + lax.axis_index("c")
        base = wid * b_per_w
        pltpu.sync_copy(idx_hbm.at[pl.ds(2 * base, 2 * b_per_w)], ia_v)

        lane = lax.iota(jnp.int32, _LANES)
        c_ev = (lane * 2) % _LANES
        c_od = (lane * 2 + 1) % _LANES
        low = lane < (_LANES // 2)

        pending = []
        for c in range(_NCHUNK):
            lo = c * rows_c
            v0 = ia_v[pl.ds(2 * lo, _LANES)]
            v1 = ia_v[pl.ds(2 * lo + _LANES, _LANES)]
            i0_v[pl.ds(lo, rows_c)] = jnp.where(
                low, _vreg_gather(v0, c_ev), _vreg_gather(v1, c_ev))
            i1_v[pl.ds(lo, rows_c)] = jnp.where(
                low, _vreg_gather(v0, c_od), _vreg_gather(v1, c_od))
            g0 = pltpu.async_copy(f0_hbm.at[i0_v.at[pl.ds(lo, rows_c)]],
                                  r0_v.at[pl.ds(lo, rows_c)], g_sems[2 * c])
            g1 = pltpu.async_copy(f1_hbm.at[i1_v.at[pl.ds(lo, rows_c)]],
                                  r1_v.at[pl.ds(lo, rows_c)], g_sems[2 * c + 1])
            pending.append((g0, g1))

        writes = []
        for c in range(_NCHUNK):
            lo = c * rows_c
            pending[c][0].wait()
            pending[c][1].wait()

            def row_body(r, carry):
                for j in range(K // _LANES):
                    sl = pl.ds(j * _LANES, _LANES)
                    r0_v[r, sl] = r0_v[r, sl] * r1_v[r, sl]
                return carry

            lax.fori_loop(lo, lo + rows_c, row_body, 0, unroll=2)
            writes.append(pltpu.async_copy(
                r0_v.at[pl.ds(lo, rows_c)],
                out_hbm.at[pl.ds(base + lo, rows_c)], w_sems[c]))
        for w in writes:
            w.wait()

    return sc_kernel(idx_flat, F0, F1)


def _tc_matmul_t(prod, F2):
    B, K = prod.shape
    N = F2.shape[0]
    BLK = 1024

    def mm_body(f2_ref, p_ref, o_ref):
        o_ref[...] = lax.dot_general(
            f2_ref[...], p_ref[...],
            (((1,), (1,)), ((), ())),
            preferred_element_type=jnp.float32,
        )

    return pl.pallas_call(
        mm_body,
        grid=(B // BLK,),
        in_specs=[
            pl.BlockSpec((N, K), lambda i: (0, 0)),
            pl.BlockSpec((BLK, K), lambda i: (i, 0)),
        ],
        out_specs=pl.BlockSpec((N, BLK), lambda i: (0, i)),
        out_shape=jax.ShapeDtypeStruct((N, B), jnp.float32),
    )(F2, prod)


def kernel(indices, F0, F1, F2, log_sigma):
    idx_flat = indices.astype(jnp.int32).reshape(-1)
    prod = _sc_gather_mul(F0, F1, idx_flat)
    res_t = _tc_matmul_t(prod, F2)
    return (res_t.T, log_sigma)
